# Initial kernel scaffold; baseline (speedup 1.0000x reference)
import jax
import jax.numpy as jnp
from jax.experimental import pallas as pl


def kernel(x, edge_index, W1, b1, W2, b2, W3, b3, bn_gamma, bn_beta, Wout, bout):
    raise NotImplementedError("write your pallas kernel here")


# speedup vs baseline: 14.0218x; 14.0218x over previous
import functools

import jax
import jax.numpy as jnp
from jax import lax
from jax.experimental import pallas as pl
from jax.experimental.pallas import tpu as pltpu
import jax.experimental.pallas.tpu_sc as plsc

N = 10000
E = 160000
D = 256
H = 128
NC = 2
NS = 16
NW = NC * NS
CHUNK = 128
NUM_CHUNKS = E // CHUNK
CHUNKS_PER_W = -(-NUM_CHUNKS // NW)
ZROWS = 624
BN = 1000
GRID = N // BN

_mesh = plsc.VectorSubcoreMesh(core_axis_name="c", subcore_axis_name="s")


@functools.partial(
    pl.kernel,
    out_type=jax.ShapeDtypeStruct((NC * N, 16), jnp.float32),
    mesh=_mesh,
    scratch_types=[
        pltpu.VMEM_SHARED((N, 16), jnp.float32),
        pltpu.VMEM((CHUNK,), jnp.int32),
        pltpu.VMEM((CHUNK, 16), jnp.float32),
        pltpu.VMEM((ZROWS, 16), jnp.float32),
    ],
)
def _sc_degree(dst_hbm, out_hbm, acc_sh, idx_v, ones_v, zero_v):
    c = lax.axis_index("c")
    s = lax.axis_index("s")
    w = s * NC + c

    def _fill(i, _):
        zero_v[i, :] = jnp.zeros((16,), jnp.float32)
        return 0
    lax.fori_loop(0, ZROWS, _fill, 0)

    def _fill1(i, _):
        ones_v[i, :] = jnp.full((16,), 1.0, jnp.float32)
        return 0
    lax.fori_loop(0, CHUNK, _fill1, 0)

    pltpu.sync_copy(zero_v, acc_sh.at[pl.ds(s * ZROWS, ZROWS)])

    @pl.when(s == NS - 1)
    def _():
        pltpu.sync_copy(zero_v.at[pl.ds(0, 16)],
                        acc_sh.at[pl.ds(NS * ZROWS, N - NS * ZROWS)])
    plsc.subcore_barrier()

    def _body(t, _):
        chunk = w + NW * t

        @pl.when(chunk < NUM_CHUNKS)
        def _():
            pltpu.sync_copy(dst_hbm.at[pl.ds(chunk * CHUNK, CHUNK)], idx_v)
            pltpu.sync_copy(ones_v, acc_sh.at[idx_v], add=True)
        return 0
    lax.fori_loop(0, CHUNKS_PER_W, _body, 0)

    plsc.subcore_barrier()

    @pl.when(s == 0)
    def _():
        pltpu.sync_copy(acc_sh, out_hbm.at[pl.ds(c * N, N)])


@functools.partial(
    pl.kernel,
    out_type=jax.ShapeDtypeStruct((NC * N, H), jnp.float32),
    mesh=_mesh,
    scratch_types=[
        pltpu.VMEM_SHARED((N, H), jnp.float32),
        pltpu.VMEM((CHUNK,), jnp.int32),
        pltpu.VMEM((CHUNK,), jnp.int32),
        pltpu.VMEM((CHUNK,), jnp.int32),
        pltpu.VMEM((CHUNK, H), jnp.float32),
        pltpu.VMEM((208, H), jnp.float32),
        pltpu.SemaphoreType.DMA,
    ],
)
def _sc_aggregate(g_hbm, src_hbm, dst_hbm, out_hbm,
                  acc_sh, sidx_v, gidx_v, didx_v, rows_v, zero_v, sem):
    c = lax.axis_index("c")
    s = lax.axis_index("s")
    w = s * NC + c

    def _fill(i, _):
        def _fill_row(j, _):
            zero_v[i, pl.ds(j * 16, 16)] = jnp.zeros((16,), jnp.float32)
            return 0
        lax.fori_loop(0, H // 16, _fill_row, 0)
        return 0
    lax.fori_loop(0, 208, _fill, 0)

    def _zero(r, _):
        pltpu.sync_copy(zero_v, acc_sh.at[pl.ds(s * ZROWS + r * 208, 208)])
        return 0
    lax.fori_loop(0, ZROWS // 208, _zero, 0)

    @pl.when(s == NS - 1)
    def _():
        pltpu.sync_copy(zero_v.at[pl.ds(0, 16)],
                        acc_sh.at[pl.ds(NS * ZROWS, N - NS * ZROWS)])
    plsc.subcore_barrier()

    def _body(t, _):
        chunk = w + NW * t

        @pl.when(chunk < NUM_CHUNKS)
        def _():
            pltpu.sync_copy(src_hbm.at[pl.ds(chunk * CHUNK, CHUNK)], sidx_v)
            pltpu.sync_copy(dst_hbm.at[pl.ds(chunk * CHUNK, CHUNK)], didx_v)

            def _shift(j, _):
                v = sidx_v[pl.ds(j * 16, 16)]
                gidx_v[pl.ds(j * 16, 16)] = v + c * N
                return 0
            lax.fori_loop(0, CHUNK // 16, _shift, 0)

            pltpu.async_copy(g_hbm.at[gidx_v], rows_v, sem).wait()
            pltpu.sync_copy(rows_v, acc_sh.at[didx_v], add=True)
        return 0
    lax.fori_loop(0, CHUNKS_PER_W, _body, 0)

    plsc.subcore_barrier()

    @pl.when(s == 0)
    def _():
        pltpu.sync_copy(acc_sh, out_hbm.at[pl.ds(c * N, N)])


def _dis_block(dp_ref):
    deg = dp_ref[0, :, 0:1] + dp_ref[1, :, 0:1] + 1.0
    return lax.rsqrt(deg)


def _tc_first_body(x_ref, w_ref, dp_ref, g_ref):
    dis = _dis_block(dp_ref)
    g = jnp.dot(x_ref[...], w_ref[...], preferred_element_type=jnp.float32) * dis
    g_ref[0] = g[:, :H]
    g_ref[1] = g[:, H:]


def _tc_mid_body(a_ref, g_ref, dp_ref, b_ref, w_ref, h_ref, gn_ref):
    dis = _dis_block(dp_ref)
    hl = jnp.maximum((a_ref[0] + g_ref[0]) * dis + b_ref[0, :H], 0.0)
    hr = jnp.maximum((a_ref[1] + g_ref[1]) * dis + b_ref[0, H:], 0.0)
    h = jnp.concatenate([hl, hr], axis=1)
    h_ref[...] = h
    gn = jnp.dot(h, w_ref[...], preferred_element_type=jnp.float32) * dis
    gn_ref[0] = gn[:, :H]
    gn_ref[1] = gn[:, H:]


def _tc_last_body(a_ref, g_ref, dp_ref, b_ref, h_ref):
    dis = _dis_block(dp_ref)
    hl = jnp.maximum((a_ref[0] + g_ref[0]) * dis + b_ref[0, :H], 0.0)
    hr = jnp.maximum((a_ref[1] + g_ref[1]) * dis + b_ref[0, H:], 0.0)
    h_ref[...] = jnp.concatenate([hl, hr], axis=1)


def _tc_stats_body(x_ref, h1_ref, h2_ref, h3_ref, s_ref):
    @pl.when(pl.program_id(0) == 0)
    def _():
        s_ref[...] = jnp.zeros_like(s_ref)
    px = jnp.concatenate(
        [x_ref[...], h1_ref[...], h2_ref[...], h3_ref[...]], axis=1)
    s = jnp.sum(px, axis=0, keepdims=True)
    ss = jnp.sum(px * px, axis=0, keepdims=True)
    s_ref[0:1, :] = s_ref[0:1, :] + s
    s_ref[1:2, :] = s_ref[1:2, :] + ss


def _tc_out_body(x_ref, h1_ref, h2_ref, h3_ref, s_ref, gam_ref, bet_ref,
                 wo_ref, bo_ref, o_ref):
    n = jnp.float32(N)
    mean = s_ref[0:1, :] / n
    var = s_ref[1:2, :] / n - mean * mean
    a = gam_ref[...] * lax.rsqrt(var + 1e-5)
    cvec = bet_ref[...] - mean * a
    px = jnp.concatenate(
        [x_ref[...], h1_ref[...], h2_ref[...], h3_ref[...]], axis=1)
    o = jnp.dot(px * a, wo_ref[...], preferred_element_type=jnp.float32)
    o = o + jnp.dot(cvec, wo_ref[...], preferred_element_type=jnp.float32)
    o_ref[...] = o + bo_ref[...]


_rb = lambda i: (i, 0)
_full = lambda i: (0, 0)
_half = lambda i: (0, i, 0)

_spec_x = pl.BlockSpec((BN, D), _rb)
_spec_w = pl.BlockSpec((D, D), _full)
_spec_dp = pl.BlockSpec((2, BN, 16), _half)
_spec_g = pl.BlockSpec((2, BN, H), _half)
_spec_b = pl.BlockSpec((1, D), _full)


def _tc_first(x, W, dp):
    return pl.pallas_call(
        _tc_first_body,
        grid=(GRID,),
        in_specs=[_spec_x, _spec_w, _spec_dp],
        out_specs=_spec_g,
        out_shape=jax.ShapeDtypeStruct((2, N, H), jnp.float32),
    )(x, W, dp)


def _tc_mid(a, g, dp, b, Wn):
    return pl.pallas_call(
        _tc_mid_body,
        grid=(GRID,),
        in_specs=[_spec_g, _spec_g, _spec_dp, _spec_b, _spec_w],
        out_specs=[_spec_x, _spec_g],
        out_shape=[jax.ShapeDtypeStruct((N, D), jnp.float32),
                   jax.ShapeDtypeStruct((2, N, H), jnp.float32)],
    )(a, g, dp, b, Wn)


def _tc_last(a, g, dp, b):
    return pl.pallas_call(
        _tc_last_body,
        grid=(GRID,),
        in_specs=[_spec_g, _spec_g, _spec_dp, _spec_b],
        out_specs=_spec_x,
        out_shape=jax.ShapeDtypeStruct((N, D), jnp.float32),
    )(a, g, dp, b)


def _tc_stats(x, h1, h2, h3):
    return pl.pallas_call(
        _tc_stats_body,
        grid=(GRID,),
        in_specs=[_spec_x] * 4,
        out_specs=pl.BlockSpec((8, 4 * D), _full),
        out_shape=jax.ShapeDtypeStruct((8, 4 * D), jnp.float32),
    )(x, h1, h2, h3)


def _tc_out(x, h1, h2, h3, sums, gamma, beta, Wout, bout):
    return pl.pallas_call(
        _tc_out_body,
        grid=(GRID,),
        in_specs=[_spec_x] * 4 + [
            pl.BlockSpec((8, 4 * D), _full),
            pl.BlockSpec((1, 4 * D), _full),
            pl.BlockSpec((1, 4 * D), _full),
            pl.BlockSpec((4 * D, D), _full),
            pl.BlockSpec((1, D), _full),
        ],
        out_specs=_spec_x,
        out_shape=jax.ShapeDtypeStruct((N, D), jnp.float32),
    )(x, h1, h2, h3, sums, gamma, beta, Wout, bout)


def kernel(x, edge_index, W1, b1, W2, b2, W3, b3, bn_gamma, bn_beta,
           Wout, bout):
    ei = edge_index.astype(jnp.int32)
    src = ei[0]
    dst = ei[1]

    dflat = _sc_degree(dst)
    dp = dflat.reshape(NC, N, 16)

    b1r = b1.reshape(1, D)
    b2r = b2.reshape(1, D)
    b3r = b3.reshape(1, D)
    gamr = bn_gamma.reshape(1, 4 * D)
    betr = bn_beta.reshape(1, 4 * D)
    bor = bout.reshape(1, D)

    g1 = _tc_first(x, W1, dp)
    a1 = _sc_aggregate(g1.reshape(NC * N, H), src, dst).reshape(NC, N, H)
    h1, g2 = _tc_mid(a1, g1, dp, b1r, W2)
    a2 = _sc_aggregate(g2.reshape(NC * N, H), src, dst).reshape(NC, N, H)
    h2, g3 = _tc_mid(a2, g2, dp, b2r, W3)
    a3 = _sc_aggregate(g3.reshape(NC * N, H), src, dst).reshape(NC, N, H)
    h3 = _tc_last(a3, g3, dp, b3r)

    sums = _tc_stats(x, h1, h2, h3)
    return _tc_out(x, h1, h2, h3, sums, gamr, betr, Wout, bor)
